# baseline (speedup 1.0000x reference)
import functools

import jax
import jax.numpy as jnp
from jax import lax
from jax.experimental import pallas as pl
from jax.experimental.pallas import tpu as pltpu, tpu_sc as plsc

NUM_CHAMPIONS = 100000
BATCH = 16384
TEAM = 5
NUM_WORKERS = 32
ROWS_PER_WORKER = BATCH // NUM_WORKERS
FLAT_PER_WORKER = ROWS_PER_WORKER * 2 * TEAM
GCHUNK = 128
NCHUNKS = FLAT_PER_WORKER // GCHUNK
LANES = 16
CHUNKS = ROWS_PER_WORKER // LANES


@functools.partial(
    pl.kernel,
    mesh=plsc.VectorSubcoreMesh(core_axis_name="c", subcore_axis_name="s"),
    out_type=jax.ShapeDtypeStruct((BATCH,), jnp.float32),
    compiler_params=pltpu.CompilerParams(needs_layout_passes=False),
    scratch_types=[
        pltpu.VMEM((2 * TEAM, ROWS_PER_WORKER), jnp.int32),
        pltpu.VMEM((FLAT_PER_WORKER,), jnp.float32),
        pltpu.VMEM((ROWS_PER_WORKER,), jnp.float32),
        pltpu.SemaphoreType.DMA,
    ],
)
def _pam_kernel(table_hbm, idx_hbm, out_hbm, idx_v, vals_v, out_v, sem):
    wid = lax.axis_index("s") * 2 + lax.axis_index("c")
    base = wid * ROWS_PER_WORKER

    pltpu.sync_copy(idx_hbm.at[:, pl.ds(base, ROWS_PER_WORKER)], idx_v)

    handles = []
    for t in range(2 * TEAM):
        for g in range(ROWS_PER_WORKER // GCHUNK):
            j = t * (ROWS_PER_WORKER // GCHUNK) + g
            handles.append(pltpu.async_copy(
                table_hbm.at[idx_v.at[t, pl.ds(g * GCHUNK, GCHUNK)]],
                vals_v.at[pl.ds(j * GCHUNK, GCHUNK)], sem))
    for h in handles:
        h.wait()

    for i in range(CHUNKS):
        sl0 = pl.ds(i * LANES, LANES)
        acc = vals_v[sl0]
        for t in range(1, TEAM):
            acc = acc + vals_v[pl.ds(t * ROWS_PER_WORKER + i * LANES, LANES)]
        for t in range(TEAM, 2 * TEAM):
            acc = acc - vals_v[pl.ds(t * ROWS_PER_WORKER + i * LANES, LANES)]
        out_v[sl0] = 1.0 / (1.0 + jnp.exp(-acc))

    pltpu.sync_copy(out_v, out_hbm.at[pl.ds(base, ROWS_PER_WORKER)])


def kernel(red, blue, strengths):
    idx = jnp.concatenate([red.T, blue.T], axis=0).astype(jnp.int32)
    return _pam_kernel(strengths, idx)


# speedup vs baseline: 1.0004x; 1.0004x over previous
import functools

import jax
import jax.numpy as jnp
from jax import lax
from jax.experimental import pallas as pl
from jax.experimental.pallas import tpu as pltpu, tpu_sc as plsc

NUM_CHAMPIONS = 100000
BATCH = 16384
TEAM = 5
NUM_WORKERS = 32
ROWS_PER_WORKER = BATCH // NUM_WORKERS
LANES = 16
CHUNKS = ROWS_PER_WORKER // LANES


@functools.partial(
    pl.kernel,
    mesh=plsc.VectorSubcoreMesh(core_axis_name="c", subcore_axis_name="s"),
    out_type=jax.ShapeDtypeStruct((BATCH,), jnp.float32),
    compiler_params=pltpu.CompilerParams(needs_layout_passes=False),
    scratch_types=[
        pltpu.VMEM((2 * TEAM, ROWS_PER_WORKER), jnp.int32),
        pltpu.VMEM((2 * TEAM, ROWS_PER_WORKER), jnp.float32),
        pltpu.VMEM((ROWS_PER_WORKER,), jnp.float32),
        pltpu.SemaphoreType.DMA,
    ],
)
def _pam_kernel(table_hbm, idx_hbm, out_hbm, idx_v, vals_v, out_v, sem):
    wid = lax.axis_index("s") * 2 + lax.axis_index("c")
    base = wid * ROWS_PER_WORKER

    pltpu.sync_copy(idx_hbm.at[:, pl.ds(base, ROWS_PER_WORKER)], idx_v)

    handles = []
    for t in range(2 * TEAM):
        for g in range(ROWS_PER_WORKER // 128):
            handles.append(pltpu.async_copy(
                table_hbm.at[idx_v.at[t, pl.ds(g * 128, 128)]],
                vals_v.at[t, pl.ds(g * 128, 128)], sem))
    for h in handles:
        h.wait()

    for i in range(CHUNKS):
        sl0 = pl.ds(i * LANES, LANES)
        acc = vals_v[0, sl0]
        for t in range(1, TEAM):
            acc = acc + vals_v[t, sl0]
        for t in range(TEAM, 2 * TEAM):
            acc = acc - vals_v[t, sl0]
        out_v[sl0] = 1.0 / (1.0 + jnp.exp(-acc))

    pltpu.sync_copy(out_v, out_hbm.at[pl.ds(base, ROWS_PER_WORKER)])


def kernel(red, blue, strengths):
    idx = jnp.concatenate([red.T, blue.T], axis=0).astype(jnp.int32)
    return _pam_kernel(strengths, idx)
